# baseline (speedup 1.0000x reference)
import functools

import jax
import jax.numpy as jnp
from jax import lax
from jax.experimental import pallas as pl
from jax.experimental.pallas import tpu as pltpu
from jax.experimental.pallas import tpu_sc as plsc

N_NODES = 100000
NUM_TYPES = 11
MAP_LEN = 35

NW = 32
ROWS_W = 3136
ROWS_LAST = N_NODES - (NW - 1) * ROWS_W
GROUPS_FULL = ROWS_W // 16
GROUPS_LAST = ROWS_LAST // 16

TC_BLOCK = 4000

_mesh = plsc.VectorSubcoreMesh(core_axis_name="c", subcore_axis_name="s")


@functools.partial(
    pl.kernel,
    mesh=_mesh,
    compiler_params=pltpu.CompilerParams(needs_layout_passes=False),
    out_type=jax.ShapeDtypeStruct((N_NODES,), jnp.int32),
    scratch_types=[
        pltpu.VMEM((ROWS_W,), jnp.int32),
        pltpu.VMEM((MAP_LEN,), jnp.int32),
        pltpu.VMEM((ROWS_W,), jnp.int32),
    ],
)
def _lookup_sc(nt_hbm, map_hbm, idx_hbm, nt_v, map_v, idx_v):
    c = lax.axis_index("c")
    s = lax.axis_index("s")
    wid = s * 2 + c
    base = wid * ROWS_W
    last = wid == NW - 1

    pltpu.sync_copy(map_hbm, map_v)

    @pl.when(jnp.logical_not(last))
    def _():
        pltpu.sync_copy(nt_hbm.at[pl.ds(base, ROWS_W)], nt_v)

    @pl.when(last)
    def _():
        pltpu.sync_copy(
            nt_hbm.at[pl.ds(base, ROWS_LAST)], nt_v.at[pl.ds(0, ROWS_LAST)]
        )

    groups = jnp.where(last, GROUPS_LAST, GROUPS_FULL)

    def body(g, carry):
        nt16 = nt_v[pl.ds(g * 16, 16)]
        idx_v[pl.ds(g * 16, 16)] = plsc.load_gather(map_v, [nt16])
        return carry

    lax.fori_loop(0, groups, body, 0)

    @pl.when(jnp.logical_not(last))
    def _():
        pltpu.sync_copy(idx_v, idx_hbm.at[pl.ds(base, ROWS_W)])

    @pl.when(last)
    def _():
        pltpu.sync_copy(
            idx_v.at[pl.ds(0, ROWS_LAST)], idx_hbm.at[pl.ds(base, ROWS_LAST)]
        )


def _expand_tc(idx_ref, out_ref):
    j = lax.broadcasted_iota(jnp.int32, (TC_BLOCK, NUM_TYPES), 1)
    out_ref[...] = jnp.where(idx_ref[...] == j, jnp.float32(0.25), jnp.float32(0.0))


def kernel(node_type, pos, mapper):
    del pos
    nt = node_type.reshape(-1)
    idx = _lookup_sc(nt, mapper)
    return pl.pallas_call(
        _expand_tc,
        grid=(N_NODES // TC_BLOCK,),
        in_specs=[pl.BlockSpec((TC_BLOCK, 1), lambda i: (i, 0))],
        out_specs=pl.BlockSpec((TC_BLOCK, NUM_TYPES), lambda i: (i, 0)),
        out_shape=jax.ShapeDtypeStruct((N_NODES, NUM_TYPES), jnp.float32),
    )(idx.reshape(N_NODES, 1))


# speedup vs baseline: 1.8009x; 1.8009x over previous
import functools

import jax
import jax.numpy as jnp
from jax import lax
from jax.experimental import pallas as pl
from jax.experimental.pallas import tpu as pltpu
from jax.experimental.pallas import tpu_sc as plsc

N_NODES = 100000
NUM_TYPES = 11
MAP_LEN = 35

IDX_PAD = 114688

NW = 32
ROWS_W = 3136
ROWS_LAST = N_NODES - (NW - 1) * ROWS_W
GROUPS_FULL = ROWS_W // 16
GROUPS_LAST = ROWS_LAST // 16

TC_ROWS = 16384
TC_GRID = 7

_mesh = plsc.VectorSubcoreMesh(core_axis_name="c", subcore_axis_name="s")


@functools.partial(
    pl.kernel,
    mesh=_mesh,
    compiler_params=pltpu.CompilerParams(needs_layout_passes=False),
    out_type=jax.ShapeDtypeStruct((IDX_PAD,), jnp.int32),
    scratch_types=[
        pltpu.VMEM((ROWS_W,), jnp.int32),
        pltpu.VMEM((MAP_LEN,), jnp.int32),
        pltpu.VMEM((ROWS_W,), jnp.int32),
    ],
)
def _lookup_sc(nt_hbm, map_hbm, idx_hbm, nt_v, map_v, idx_v):
    c = lax.axis_index("c")
    s = lax.axis_index("s")
    wid = s * 2 + c
    base = wid * ROWS_W
    last = wid == NW - 1

    pltpu.sync_copy(map_hbm, map_v)

    @pl.when(jnp.logical_not(last))
    def _():
        pltpu.sync_copy(nt_hbm.at[pl.ds(base, ROWS_W)], nt_v)

    @pl.when(last)
    def _():
        pltpu.sync_copy(
            nt_hbm.at[pl.ds(base, ROWS_LAST)], nt_v.at[pl.ds(0, ROWS_LAST)]
        )

    groups = jnp.where(last, GROUPS_LAST, GROUPS_FULL)

    def body(g, carry):
        nt16 = nt_v[pl.ds(g * 16, 16)]
        idx_v[pl.ds(g * 16, 16)] = plsc.load_gather(map_v, [nt16])
        return carry

    lax.fori_loop(0, groups, body, 0)

    @pl.when(jnp.logical_not(last))
    def _():
        pltpu.sync_copy(idx_v, idx_hbm.at[pl.ds(base, ROWS_W)])

    @pl.when(last)
    def _():
        pltpu.sync_copy(
            idx_v.at[pl.ds(0, ROWS_LAST)], idx_hbm.at[pl.ds(base, ROWS_LAST)]
        )


def _expand_tc(idx_ref, out_ref):
    idxb = idx_ref[...]
    t = jnp.transpose(idxb, (1, 0))
    j = lax.broadcasted_iota(jnp.int32, (128, NUM_TYPES), 1)
    quarter = jnp.float32(0.25)
    zero = jnp.float32(0.0)
    for r in range(128):
        col = t[:, r : r + 1]
        out_ref[pl.ds(r * 128, 128), :] = jnp.where(col == j, quarter, zero)


def kernel(node_type, pos, mapper):
    del pos
    nt = node_type.reshape(-1)
    idx = _lookup_sc(nt, mapper)
    return pl.pallas_call(
        _expand_tc,
        grid=(TC_GRID,),
        in_specs=[pl.BlockSpec((128, 128), lambda i: (i, 0))],
        out_specs=pl.BlockSpec((TC_ROWS, NUM_TYPES), lambda i: (i, 0)),
        out_shape=jax.ShapeDtypeStruct((N_NODES, NUM_TYPES), jnp.float32),
    )(idx.reshape(IDX_PAD // 128, 128))


# speedup vs baseline: 1.8706x; 1.0387x over previous
import functools

import jax
import jax.numpy as jnp
from jax import lax
from jax.experimental import pallas as pl
from jax.experimental.pallas import tpu as pltpu
from jax.experimental.pallas import tpu_sc as plsc

N_NODES = 100000
NUM_TYPES = 11
MAP_LEN = 35

IDX_PAD = 114688

NW = 32
ROWS_W = 3136
ROWS_LAST = N_NODES - (NW - 1) * ROWS_W
GROUPS_FULL = ROWS_W // 16
GROUPS_LAST = ROWS_LAST // 16

TC_ROWS = 16384
TC_GRID = 7

_mesh = plsc.VectorSubcoreMesh(core_axis_name="c", subcore_axis_name="s")


@functools.partial(
    pl.kernel,
    mesh=_mesh,
    compiler_params=pltpu.CompilerParams(needs_layout_passes=False),
    out_type=jax.ShapeDtypeStruct((IDX_PAD,), jnp.int32),
    scratch_types=[
        pltpu.VMEM((ROWS_W,), jnp.int32),
        pltpu.VMEM((MAP_LEN,), jnp.int32),
        pltpu.VMEM((ROWS_W,), jnp.int32),
    ],
)
def _lookup_sc(nt_hbm, map_hbm, idx_hbm, nt_v, map_v, idx_v):
    c = lax.axis_index("c")
    s = lax.axis_index("s")
    wid = s * 2 + c
    base = wid * ROWS_W
    last = wid == NW - 1

    pltpu.sync_copy(map_hbm, map_v)

    @pl.when(jnp.logical_not(last))
    def _():
        pltpu.sync_copy(nt_hbm.at[pl.ds(base, ROWS_W)], nt_v)

    @pl.when(last)
    def _():
        pltpu.sync_copy(
            nt_hbm.at[pl.ds(base, ROWS_LAST)], nt_v.at[pl.ds(0, ROWS_LAST)]
        )

    groups = jnp.where(last, GROUPS_LAST, GROUPS_FULL)

    def body(g, carry):
        nt16 = nt_v[pl.ds(g * 16, 16)]
        idx_v[pl.ds(g * 16, 16)] = plsc.load_gather(map_v, [nt16])
        return carry

    lax.fori_loop(0, groups, body, 0)

    @pl.when(jnp.logical_not(last))
    def _():
        pltpu.sync_copy(idx_v, idx_hbm.at[pl.ds(base, ROWS_W)])

    @pl.when(last)
    def _():
        pltpu.sync_copy(
            idx_v.at[pl.ds(0, ROWS_LAST)], idx_hbm.at[pl.ds(base, ROWS_LAST)]
        )


def _expand_tc(idx_ref, out_ref):
    idxb = idx_ref[...]
    c_col = lax.broadcasted_iota(jnp.int32, (NUM_TYPES, 128), 0)
    rows = lax.broadcasted_iota(jnp.int32, (NUM_TYPES, NUM_TYPES), 0)
    cols = lax.broadcasted_iota(jnp.int32, (NUM_TYPES, NUM_TYPES), 1)
    eye = (rows == cols).astype(jnp.float32)
    quarter = jnp.float32(0.25)
    zero = jnp.float32(0.0)
    for r in range(128):
        xr = idxb[r : r + 1, :]
        onehot_t = jnp.where(xr == c_col, quarter, zero)
        res = lax.dot_general(
            onehot_t, eye, (((0,), (0,)), ((), ())),
            preferred_element_type=jnp.float32,
        )
        out_ref[pl.ds(r * 128, 128), :] = res


def kernel(node_type, pos, mapper):
    del pos
    nt = node_type.reshape(-1)
    idx = _lookup_sc(nt, mapper)
    return pl.pallas_call(
        _expand_tc,
        grid=(TC_GRID,),
        in_specs=[pl.BlockSpec((128, 128), lambda i: (i, 0))],
        out_specs=pl.BlockSpec((TC_ROWS, NUM_TYPES), lambda i: (i, 0)),
        out_shape=jax.ShapeDtypeStruct((N_NODES, NUM_TYPES), jnp.float32),
    )(idx.reshape(IDX_PAD // 128, 128))


# speedup vs baseline: 1.8720x; 1.0008x over previous
import functools

import jax
import jax.numpy as jnp
from jax import lax
from jax.experimental import pallas as pl
from jax.experimental.pallas import tpu as pltpu
from jax.experimental.pallas import tpu_sc as plsc

N_NODES = 100000
NUM_TYPES = 11
MAP_LEN = 35

IDX_PAD = 114688

NW = 32
ROWS_W = 3136
ROWS_LAST = N_NODES - (NW - 1) * ROWS_W
GROUPS_FULL = ROWS_W // 16
GROUPS_LAST = ROWS_LAST // 16

TC_ROWS = 16384
TC_GRID = 7

_mesh = plsc.VectorSubcoreMesh(core_axis_name="c", subcore_axis_name="s")


@functools.partial(
    pl.kernel,
    mesh=_mesh,
    compiler_params=pltpu.CompilerParams(needs_layout_passes=False),
    out_type=jax.ShapeDtypeStruct((IDX_PAD,), jnp.int32),
    scratch_types=[
        pltpu.VMEM((ROWS_W,), jnp.int32),
        pltpu.VMEM((MAP_LEN,), jnp.int32),
        pltpu.VMEM((ROWS_W,), jnp.int32),
    ],
)
def _lookup_sc(nt_hbm, map_hbm, idx_hbm, nt_v, map_v, idx_v):
    c = lax.axis_index("c")
    s = lax.axis_index("s")
    wid = s * 2 + c
    base = wid * ROWS_W
    last = wid == NW - 1

    pltpu.sync_copy(map_hbm, map_v)

    @pl.when(jnp.logical_not(last))
    def _():
        pltpu.sync_copy(nt_hbm.at[pl.ds(base, ROWS_W)], nt_v)

    @pl.when(last)
    def _():
        pltpu.sync_copy(
            nt_hbm.at[pl.ds(base, ROWS_LAST)], nt_v.at[pl.ds(0, ROWS_LAST)]
        )

    @pl.when(last)
    def _():
        zpad = jnp.zeros((16,), jnp.int32)
        nt_v[pl.ds(ROWS_LAST, 16)] = zpad
        nt_v[pl.ds(ROWS_LAST + 16, 16)] = zpad

    groups = jnp.where(last, (ROWS_LAST + 63) // 64, ROWS_W // 64)

    def body(g, carry):
        for u in range(4):
            o = g * 64 + u * 16
            nt16 = nt_v[pl.ds(o, 16)]
            idx_v[pl.ds(o, 16)] = plsc.load_gather(map_v, [nt16])
        return carry

    lax.fori_loop(0, groups, body, 0)

    @pl.when(jnp.logical_not(last))
    def _():
        pltpu.sync_copy(idx_v, idx_hbm.at[pl.ds(base, ROWS_W)])

    @pl.when(last)
    def _():
        pltpu.sync_copy(
            idx_v.at[pl.ds(0, ROWS_LAST)], idx_hbm.at[pl.ds(base, ROWS_LAST)]
        )


def _expand_tc(idx_ref, out_ref):
    idxb = idx_ref[...]
    c_col = lax.broadcasted_iota(jnp.int32, (NUM_TYPES, 128), 0)
    rows = lax.broadcasted_iota(jnp.int32, (NUM_TYPES, NUM_TYPES), 0)
    cols = lax.broadcasted_iota(jnp.int32, (NUM_TYPES, NUM_TYPES), 1)
    eye = (rows == cols).astype(jnp.float32)
    quarter = jnp.float32(0.25)
    zero = jnp.float32(0.0)
    for r in range(128):
        xr = idxb[r : r + 1, :]
        onehot_t = jnp.where(xr == c_col, quarter, zero)
        res = lax.dot_general(
            onehot_t, eye, (((0,), (0,)), ((), ())),
            preferred_element_type=jnp.float32,
        )
        out_ref[pl.ds(r * 128, 128), :] = res


def kernel(node_type, pos, mapper):
    del pos
    nt = node_type.reshape(-1)
    idx = _lookup_sc(nt, mapper)
    return pl.pallas_call(
        _expand_tc,
        grid=(TC_GRID,),
        in_specs=[pl.BlockSpec((128, 128), lambda i: (i, 0))],
        out_specs=pl.BlockSpec((TC_ROWS, NUM_TYPES), lambda i: (i, 0)),
        out_shape=jax.ShapeDtypeStruct((N_NODES, NUM_TYPES), jnp.float32),
    )(idx.reshape(IDX_PAD // 128, 128))


# speedup vs baseline: 1.9133x; 1.0221x over previous
import functools

import jax
import jax.numpy as jnp
from jax import lax
from jax.experimental import pallas as pl
from jax.experimental.pallas import tpu as pltpu
from jax.experimental.pallas import tpu_sc as plsc

N_NODES = 100000
NUM_TYPES = 11
MAP_LEN = 35

NW = 32
ROWS_W = 3136
NT_PAD = NW * ROWS_W
IDX_PAD = 114688

TC_ROWS = 16384
TC_GRID = 7

_mesh = plsc.VectorSubcoreMesh(core_axis_name="c", subcore_axis_name="s")


@functools.partial(
    pl.kernel,
    mesh=_mesh,
    compiler_params=pltpu.CompilerParams(needs_layout_passes=False),
    out_type=jax.ShapeDtypeStruct((IDX_PAD,), jnp.int32),
    scratch_types=[
        pltpu.VMEM((ROWS_W,), jnp.int32),
        pltpu.VMEM((MAP_LEN,), jnp.int32),
        pltpu.VMEM((ROWS_W,), jnp.int32),
        pltpu.SemaphoreType.DMA,
        pltpu.SemaphoreType.DMA,
    ],
)
def _lookup_sc(nt_hbm, map_hbm, idx_hbm, nt_v, map_v, idx_v, sem_m, sem_n):
    c = lax.axis_index("c")
    s = lax.axis_index("s")
    wid = s * 2 + c
    base = wid * ROWS_W

    cp_m = pltpu.async_copy(map_hbm, map_v, sem_m)
    cp_n = pltpu.async_copy(nt_hbm.at[pl.ds(base, ROWS_W)], nt_v, sem_n)
    cp_m.wait()
    cp_n.wait()

    def body(g, carry):
        for u in range(4):
            o = g * 64 + u * 16
            nt16 = nt_v[pl.ds(o, 16)]
            idx_v[pl.ds(o, 16)] = plsc.load_gather(map_v, [nt16])
        return carry

    lax.fori_loop(0, ROWS_W // 64, body, 0)

    pltpu.sync_copy(idx_v, idx_hbm.at[pl.ds(base, ROWS_W)])


def _expand_tc(idx_ref, out_ref):
    idxb = idx_ref[...]
    c_col = lax.broadcasted_iota(jnp.int32, (NUM_TYPES, 128), 0)
    rows = lax.broadcasted_iota(jnp.int32, (NUM_TYPES, NUM_TYPES), 0)
    cols = lax.broadcasted_iota(jnp.int32, (NUM_TYPES, NUM_TYPES), 1)
    eye = (rows == cols).astype(jnp.float32)
    quarter = jnp.float32(0.25)
    zero = jnp.float32(0.0)
    for r in range(128):
        xr = idxb[r : r + 1, :]
        onehot_t = jnp.where(xr == c_col, quarter, zero)
        res = lax.dot_general(
            onehot_t, eye, (((0,), (0,)), ((), ())),
            preferred_element_type=jnp.float32,
        )
        out_ref[pl.ds(r * 128, 128), :] = res


def kernel(node_type, pos, mapper):
    del pos
    nt = jnp.pad(node_type.reshape(-1), (0, NT_PAD - N_NODES))
    idx = _lookup_sc(nt, mapper)
    return pl.pallas_call(
        _expand_tc,
        grid=(TC_GRID,),
        in_specs=[pl.BlockSpec((128, 128), lambda i: (i, 0))],
        out_specs=pl.BlockSpec((TC_ROWS, NUM_TYPES), lambda i: (i, 0)),
        out_shape=jax.ShapeDtypeStruct((N_NODES, NUM_TYPES), jnp.float32),
    )(idx.reshape(IDX_PAD // 128, 128))


# speedup vs baseline: 1.9135x; 1.0001x over previous
import functools

import jax
import jax.numpy as jnp
from jax import lax
from jax.experimental import pallas as pl
from jax.experimental.pallas import tpu as pltpu
from jax.experimental.pallas import tpu_sc as plsc

N_NODES = 100000
NUM_TYPES = 11
MAP_LEN = 35

NW = 32
ROWS_W = 3136
NT_PAD = NW * ROWS_W
IDX_PAD = 114688

TC_ROWS = 16384
TC_GRID = 7

_mesh = plsc.VectorSubcoreMesh(core_axis_name="c", subcore_axis_name="s")


@functools.partial(
    pl.kernel,
    mesh=_mesh,
    compiler_params=pltpu.CompilerParams(needs_layout_passes=False),
    out_type=jax.ShapeDtypeStruct((IDX_PAD,), jnp.int32),
    scratch_types=[
        pltpu.VMEM((ROWS_W,), jnp.int32),
        pltpu.VMEM((MAP_LEN,), jnp.int32),
        pltpu.VMEM((ROWS_W,), jnp.int32),
        pltpu.SemaphoreType.DMA,
        pltpu.SemaphoreType.DMA,
    ],
)
def _lookup_sc(nt_hbm, map_hbm, idx_hbm, nt_v, map_v, idx_v, sem_m, sem_n):
    c = lax.axis_index("c")
    s = lax.axis_index("s")
    wid = s * 2 + c
    base = wid * ROWS_W

    cp_m = pltpu.async_copy(map_hbm, map_v, sem_m)
    cp_n = pltpu.async_copy(nt_hbm.at[pl.ds(base, ROWS_W)], nt_v, sem_n)
    cp_m.wait()
    cp_n.wait()

    def body(g, carry):
        for u in range(4):
            o = g * 64 + u * 16
            nt16 = nt_v[pl.ds(o, 16)]
            idx_v[pl.ds(o, 16)] = plsc.load_gather(map_v, [nt16])
        return carry

    lax.fori_loop(0, ROWS_W // 64, body, 0)

    pltpu.sync_copy(idx_v, idx_hbm.at[pl.ds(base, ROWS_W)])


def _expand_tc(idx_ref, out_ref):
    idxb = idx_ref[...]
    c_col = lax.broadcasted_iota(jnp.int32, (NUM_TYPES, 128), 0)
    rows = lax.broadcasted_iota(jnp.int32, (NUM_TYPES, NUM_TYPES), 0)
    cols = lax.broadcasted_iota(jnp.int32, (NUM_TYPES, NUM_TYPES), 1)
    eye = (rows == cols).astype(jnp.float32)
    quarter = jnp.float32(0.25)
    zero = jnp.float32(0.0)
    for r in range(TC_ROWS // 128):
        xr = idxb[r : r + 1, :]
        onehot_t = jnp.where(xr == c_col, quarter, zero)
        res = lax.dot_general(
            onehot_t, eye, (((0,), (0,)), ((), ())),
            preferred_element_type=jnp.float32,
        )
        out_ref[pl.ds(r * 128, 128), :] = res


def kernel(node_type, pos, mapper):
    del pos
    nt = jnp.pad(node_type.reshape(-1), (0, NT_PAD - N_NODES))
    idx = _lookup_sc(nt, mapper)
    return pl.pallas_call(
        _expand_tc,
        grid=(TC_GRID,),
        in_specs=[pl.BlockSpec((TC_ROWS // 128, 128), lambda i: (i, 0))],
        out_specs=pl.BlockSpec((TC_ROWS, NUM_TYPES), lambda i: (i, 0)),
        out_shape=jax.ShapeDtypeStruct((N_NODES, NUM_TYPES), jnp.float32),
    )(idx.reshape(IDX_PAD // 128, 128))
